# baseline (speedup 1.0000x reference)
import functools

import jax
import jax.numpy as jnp
from jax import lax
from jax.experimental import pallas as pl
from jax.experimental.pallas import tpu as pltpu
from jax.experimental.pallas import tpu_sc as plsc

_KS0 = 0
_KS1 = 42
_KS2 = _KS0 ^ _KS1 ^ 0x1BD11BDA

_ROTS = ((13, 15, 26, 6), (17, 29, 16, 24))

_BLK = 65536

_NC = 2
_NS = 16
_W = _NC * _NS
_CH = 4096
_U = 4
_CHUNKS = 4
_S = _W * _CH * _CHUNKS


def _rotl(x, r):
    return (x << jnp.int32(r)) | lax.shift_right_logical(x, jnp.int32(32 - r))


def _threefry_scale(x1):
    x0 = jnp.zeros_like(x1)
    ks = (jnp.int32(_KS0), jnp.int32(_KS1), jnp.int32(_KS2))
    for i in range(5):
        for r in _ROTS[i % 2]:
            x0 = x0 + x1
            x1 = _rotl(x1, r)
            x1 = x0 ^ x1
        x0 = x0 + ks[(i + 1) % 3]
        x1 = x1 + (ks[(i + 2) % 3] + jnp.int32(i + 1))
    bits = x0 ^ x1
    return jnp.where(bits >= 0, jnp.float32(2.0), jnp.float32(0.0))


def _tc_body(iota_ref, v_ref, o_ref):
    pid = pl.program_id(0)
    base = (pid + jnp.int32(_S // _BLK)) * jnp.int32(_BLK) + jnp.int32(_KS1)
    x1 = iota_ref[...] + base
    o_ref[...] = v_ref[...] * _threefry_scale(x1)


def _tc_dropout(values, n):
    grid = pl.cdiv(n - _S, _BLK)
    iota = jnp.arange(_BLK, dtype=jnp.int32)
    off = _S // _BLK
    return pl.pallas_call(
        _tc_body,
        grid=(grid,),
        in_specs=[
            pl.BlockSpec((_BLK,), lambda i: (0,)),
            pl.BlockSpec((_BLK,), lambda i: (i + off,)),
        ],
        out_specs=pl.BlockSpec((_BLK,), lambda i: (i + off,)),
        out_shape=jax.ShapeDtypeStruct((n,), jnp.float32),
    )(iota, values)


def _sc_body(v_hbm, o_hbm, vbuf, obuf):
    wid = lax.axis_index("c") * _NS + lax.axis_index("s")
    span = _CH * _CHUNKS
    lane = lax.iota(jnp.int32, 16)
    for k in range(_CHUNKS):
        base = wid * span + k * _CH
        pltpu.sync_copy(v_hbm.at[pl.ds(base, _CH)], vbuf)

        def body(j, carry):
            for u in range(_U):
                o = j * (16 * _U) + u * 16
                x1 = lane + (base + o + jnp.int32(_KS1))
                scale = _threefry_scale(x1)
                obuf[pl.ds(o, 16)] = vbuf[pl.ds(o, 16)] * scale
            return carry

        lax.fori_loop(0, _CH // (16 * _U), body, jnp.int32(0))
        pltpu.sync_copy(obuf, o_hbm.at[pl.ds(base, _CH)])


@functools.partial(
    pl.kernel,
    out_type=jax.ShapeDtypeStruct((_S,), jnp.float32),
    mesh=plsc.VectorSubcoreMesh(core_axis_name="c", subcore_axis_name="s"),
    scratch_types=[
        pltpu.VMEM((_CH,), jnp.float32),
        pltpu.VMEM((_CH,), jnp.float32),
    ],
)
def _sc_dropout(v_hbm, o_hbm, vbuf, obuf):
    _sc_body(v_hbm, o_hbm, vbuf, obuf)


@jax.jit
def _sparse_dropout(values):
    n = values.shape[0]
    tc_full = _tc_dropout(values, n)
    sc_head = _sc_dropout(values)
    return lax.dynamic_update_slice(tc_full, sc_head, (0,))


def kernel(indices, values):
    del indices
    return _sparse_dropout(values)


# speedup vs baseline: 1.2360x; 1.2360x over previous
import jax
import jax.numpy as jnp
import numpy as np
from jax import lax
from jax.experimental import pallas as pl

_BLK = 65536

_KS0 = 0
_KS1 = 42
_KS2 = _KS0 ^ _KS1 ^ 0x1BD11BDA

_ROTS = ((13, 15, 26, 6), (17, 29, 16, 24))

_IOTA = np.arange(_BLK, dtype=np.int32)


def _rotl(x, r):
    return (x << jnp.int32(r)) | lax.shift_right_logical(x, jnp.int32(32 - r))


def _threefry_scale(x1):
    ks = (_KS0, _KS1, _KS2)
    x0 = None
    for i in range(5):
        for j, r in enumerate(_ROTS[i % 2]):
            x0 = x1 if x0 is None else x0 + x1
            x1 = x0 ^ _rotl(x1, r)
        a = ks[(i + 1) % 3]
        if a:
            x0 = x0 + jnp.int32(a)
        x1 = x1 + jnp.int32(ks[(i + 2) % 3] + i + 1)
    bits = x0 ^ x1
    return jnp.where(bits >= 0, jnp.float32(2.0), jnp.float32(0.0))


def _body(iota_ref, v_ref, o_ref):
    pid = pl.program_id(0)
    x1 = iota_ref[...] + (pid * jnp.int32(_BLK) + jnp.int32(_KS1))
    v = v_ref[...]
    o_ref[...] = v * _threefry_scale(x1)


@jax.jit
def _sparse_dropout(values):
    n = values.shape[0]
    grid = pl.cdiv(n, _BLK)
    return pl.pallas_call(
        _body,
        grid=(grid,),
        in_specs=[
            pl.BlockSpec((_BLK,), lambda i: (0,)),
            pl.BlockSpec((_BLK,), lambda i: (i,)),
        ],
        out_specs=pl.BlockSpec((_BLK,), lambda i: (i,)),
        out_shape=jax.ShapeDtypeStruct((n,), jnp.float32),
    )(_IOTA, values)


def kernel(indices, values):
    del indices
    return _sparse_dropout(values)


# speedup vs baseline: 1.2997x; 1.0516x over previous
import jax
import jax.numpy as jnp
import numpy as np
from jax import lax
from jax.experimental import pallas as pl

_BLK = 244736

_KS0 = 0
_KS1 = 42
_KS2 = _KS0 ^ _KS1 ^ 0x1BD11BDA

_ROTS = ((13, 15, 26, 6), (17, 29, 16, 24))

_IOTA = np.arange(_BLK, dtype=np.int32)


def _rotl(x, r):
    return (x << jnp.int32(r)) | lax.shift_right_logical(x, jnp.int32(32 - r))


def _threefry_scale(x1):
    ks = (_KS0, _KS1, _KS2)
    x0 = None
    for i in range(5):
        for j, r in enumerate(_ROTS[i % 2]):
            x0 = x1 if x0 is None else x0 + x1
            x1 = x0 ^ _rotl(x1, r)
        a = ks[(i + 1) % 3]
        if a:
            x0 = x0 + jnp.int32(a)
        x1 = x1 + jnp.int32(ks[(i + 2) % 3] + i + 1)
    bits = x0 ^ x1
    return jnp.where(bits >= 0, jnp.float32(2.0), jnp.float32(0.0))


def _body(iota_ref, v_ref, o_ref):
    pid = pl.program_id(0)
    x1 = iota_ref[...] + (pid * jnp.int32(_BLK) + jnp.int32(_KS1))
    v = v_ref[...]
    o_ref[...] = v * _threefry_scale(x1)


@jax.jit
def _sparse_dropout(values):
    n = values.shape[0]
    grid = pl.cdiv(n, _BLK)
    return pl.pallas_call(
        _body,
        grid=(grid,),
        in_specs=[
            pl.BlockSpec((_BLK,), lambda i: (0,)),
            pl.BlockSpec((_BLK,), lambda i: (i,)),
        ],
        out_specs=pl.BlockSpec((_BLK,), lambda i: (i,)),
        out_shape=jax.ShapeDtypeStruct((n,), jnp.float32),
    )(_IOTA, values)


def kernel(indices, values):
    del indices
    return _sparse_dropout(values)
